# baseline (speedup 1.0000x reference)
import numpy as np
import jax
import jax.numpy as jnp
from jax import lax
from jax.experimental import pallas as pl
from jax.experimental.pallas import tpu as pltpu

B = 16384
HN = 8
OBS = HN + 1
ANUM = 2 ** HN
VQ_DIM = 4
VQ_SIZE = 8
LAT = 16
NN0, NN1 = 258, 128

BB = 4096
W = VQ_SIZE * LAT

_GRP = (np.arange(LAT)[:, None] // VQ_DIM ==
        np.arange(LAT)[None, :] // VQ_DIM).astype(np.float32)
_G1 = np.tile(_GRP, (1, VQ_SIZE))
_LPR = np.kron(np.triu(np.ones((VQ_SIZE, VQ_SIZE), np.float32), 1),
               np.eye(LAT, dtype=np.float32))


def _mm(x, w):
    return jnp.dot(x, w, preferred_element_type=jnp.float32)


def _dt(x, w):
    return lax.dot_general(x, w, (((1,), (1,)), ((), ())),
                           preferred_element_type=jnp.float32)


def _dx(xt, w):
    return lax.dot_general(xt, w, (((0,), (0,)), ((), ())),
                           preferred_element_type=jnp.float32)


def _vq_block(ot_ref, a_ref, w1_ref, b1_ref, w2t_ref, b2_ref, wl_ref, bl_ref,
              cbt_ref, aw1t_ref, ab1_ref, aw2_ref, ab2_ref, awpt_ref, abp_ref,
              ow1t_ref, ob1_ref, ow2_ref, ob2_ref, owo_ref, obo_ref,
              g1_ref, lpr_ref,
              rep_ref, reot_ref, latt_ref, vqt_ref, quantt_ref):
    f32 = jnp.float32
    bf16 = jnp.bfloat16
    xot = ot_ref[...]
    a = a_ref[...]

    iota_v = lax.broadcasted_iota(jnp.int32, (ANUM, BB), 0)
    a_hott = jnp.where(iota_v == a, 1.0, 0.0).astype(f32)

    w1 = w1_ref[...]
    ht = _mm(w1[:, OBS:], a_hott) + _mm(w1[:, :OBS], xot) \
        + b1_ref[...][:, None]
    ht = jnp.maximum(ht, 0.0)
    h2t = jnp.maximum(_dx(w2t_ref[...], ht) + b2_ref[...][:, None],
                      0.0)
    latt = _mm(wl_ref[...], h2t) + bl_ref[...][:, None]
    latt_ref[...] = latt

    cbt = cbt_ref[...]
    md = -2.0 * jnp.repeat(jnp.tile(cbt, (VQ_DIM, 1)), LAT, axis=1) \
        * g1_ref[...]
    qmt = -0.125 * md
    cb2v = jnp.sum(0.25 * md * md, axis=0, keepdims=True)

    score = _dx(latt, md) + cb2v
    s = jnp.minimum(score, pltpu.roll(score, 64, 1))
    s = jnp.minimum(s, pltpu.roll(s, 32, 1))
    min_t = jnp.minimum(s, pltpu.roll(s, 16, 1))
    onehot = jnp.where(score == min_t, 1.0, 0.0)
    cnt = _mm(onehot, lpr_ref[...])
    first = jnp.where(cnt == 0.0, onehot, 0.0)
    qt = _dt(qmt, first)
    vqt = latt + (qt - latt)
    vqt_ref[...] = vqt
    quantt_ref[...] = qt

    vqbt = vqt.astype(bf16)
    xotb = xot.astype(bf16)
    aw1t = aw1t_ref[...]
    ha = _dx(vqbt, aw1t[:LAT].astype(bf16)) \
        + _dx(xotb, aw1t[LAT:].astype(bf16)) + ab1_ref[...][None, :]
    ha = jnp.maximum(ha, 0.0).astype(bf16)
    ha = jnp.maximum(_dt(ha, aw2_ref[...].astype(bf16))
                     + ab2_ref[...][None, :], 0.0).astype(bf16)
    rep_ref[...] = _mm(ha, awpt_ref[...].astype(bf16)) \
        + abp_ref[...][None, :]

    ho = jnp.maximum(_dx(vqbt, ow1t_ref[...].astype(bf16))
                     + ob1_ref[...][None, :], 0.0).astype(bf16)
    ho = jnp.maximum(_dt(ho, ow2_ref[...].astype(bf16))
                     + ob2_ref[...][None, :], 0.0).astype(bf16)
    reot_ref[...] = _dt(owo_ref[...].astype(bf16), ho) \
        + obo_ref[...][:, None]


def kernel(o, a, enc_w1, enc_b1, enc_w2, enc_b2, enc_wl, enc_bl, codebook,
           ad_w1, ad_b1, ad_w2, ad_b2, ad_wp, ad_bp,
           od_w1, od_b1, od_w2, od_b2, od_wo, od_bo):
    f32 = jnp.float32

    consts = [jnp.asarray(_G1), jnp.asarray(_LPR)]
    weights = [enc_w1, enc_b1, enc_w2.T, enc_b2, enc_wl, enc_bl,
               codebook.T, ad_w1.T, ad_b1, ad_w2, ad_b2, ad_wp.T,
               ad_bp, od_w1.T, od_b1, od_w2, od_b2, od_wo,
               od_bo] + consts

    def bcast_spec(arr):
        if arr.ndim == 1:
            return pl.BlockSpec(arr.shape, lambda i: (0,))
        return pl.BlockSpec(arr.shape, lambda i: (0, 0))

    in_specs = [
        pl.BlockSpec((OBS, BB), lambda i: (0, i)),
        pl.BlockSpec((1, BB), lambda i: (0, i)),
    ] + [bcast_spec(w) for w in weights]

    out_specs = (
        pl.BlockSpec((BB, ANUM), lambda i: (i, 0)),
        pl.BlockSpec((OBS, BB), lambda i: (0, i)),
        pl.BlockSpec((LAT, BB), lambda i: (0, i)),
        pl.BlockSpec((LAT, BB), lambda i: (0, i)),
        pl.BlockSpec((LAT, BB), lambda i: (0, i)),
    )
    out_shape = (
        jax.ShapeDtypeStruct((B, ANUM), f32),
        jax.ShapeDtypeStruct((OBS, B), f32),
        jax.ShapeDtypeStruct((LAT, B), f32),
        jax.ShapeDtypeStruct((LAT, B), f32),
        jax.ShapeDtypeStruct((LAT, B), f32),
    )

    re_p, re_ot, latentt, vq_latentt, quantizedt = pl.pallas_call(
        _vq_block,
        grid=(B // BB,),
        in_specs=in_specs,
        out_specs=out_specs,
        out_shape=out_shape,
    )(o.T, a.reshape(1, B), *weights)
    return (re_p, re_ot.T, latentt.T, vq_latentt.T, quantizedt.T)
